# Initial kernel scaffold; baseline (speedup 1.0000x reference)
import functools

import jax
import jax.numpy as jnp
from jax import lax
from jax.experimental import pallas as pl
from jax.experimental.pallas import tpu as pltpu
from jax.experimental.pallas import tpu_sc as plsc

S = 100000
DS = 32
N = 524288
DB = 26
DE = 26
H = 32
CP = 32


_S_BLK = 2000


def _mlp_body(sf, wb1, bb1, wb2, bb2, we1, be1, we2, be2, tb_out, te_out):
    x = sf[...]
    hb = jax.nn.silu(jnp.dot(x, wb1[...], preferred_element_type=jnp.float32)
                     + bb1[...])
    tb_out[...] = (jnp.dot(hb, wb2[...], preferred_element_type=jnp.float32)
                   + bb2[...])
    he = jax.nn.silu(jnp.dot(x, we1[...], preferred_element_type=jnp.float32)
                     + be1[...])
    te_out[...] = (jnp.dot(he, we2[...], preferred_element_type=jnp.float32)
                   + be2[...])


def _mlp_tables(sf, Wb1, bb1, Wb2, bb2, We1, be1, We2, be2):
    Wb2p = jnp.pad(Wb2, ((0, 0), (0, CP - DB)))
    bb2p = jnp.pad(bb2, (0, CP - DB)).reshape(1, CP)
    We2p = jnp.pad(We2, ((0, 0), (0, CP - DE)))
    be2p = jnp.pad(be2, (0, CP - DE)).reshape(1, CP)
    bb1r = bb1.reshape(1, H)
    be1r = be1.reshape(1, H)

    grid = S // _S_BLK
    full = lambda i: (0, 0)
    return pl.pallas_call(
        _mlp_body,
        grid=(grid,),
        in_specs=[
            pl.BlockSpec((_S_BLK, DS), lambda i: (i, 0)),
            pl.BlockSpec((DS, H), full),
            pl.BlockSpec((1, H), full),
            pl.BlockSpec((H, CP), full),
            pl.BlockSpec((1, CP), full),
            pl.BlockSpec((DS, H), full),
            pl.BlockSpec((1, H), full),
            pl.BlockSpec((H, CP), full),
            pl.BlockSpec((1, CP), full),
        ],
        out_specs=[
            pl.BlockSpec((_S_BLK, CP), lambda i: (i, 0)),
            pl.BlockSpec((_S_BLK, CP), lambda i: (i, 0)),
        ],
        out_shape=[
            jax.ShapeDtypeStruct((S, CP), jnp.float32),
            jax.ShapeDtypeStruct((S, CP), jnp.float32),
        ],
    )(sf, Wb1, bb1r, Wb2p, bb2p, We1, be1r, We2p, be2p)


_NC = 2
_NS = 16
_NW = _NC * _NS
_ROWS_W = N // _NW
_CHUNK = 1024
_NCHUNK = _ROWS_W // _CHUNK


def _sc_gather(loc_ind, tb, te):
    mesh = plsc.VectorSubcoreMesh(core_axis_name="c", subcore_axis_name="s")

    @functools.partial(
        pl.kernel,
        mesh=mesh,
        out_type=(
            jax.ShapeDtypeStruct((N, CP), jnp.float32),
            jax.ShapeDtypeStruct((N, CP), jnp.float32),
        ),
        scratch_types=[
            pltpu.VMEM((_ROWS_W,), jnp.int32),
            pltpu.VMEM((_CHUNK, CP), jnp.float32),
            pltpu.VMEM((_CHUNK, CP), jnp.float32),
            pltpu.SemaphoreType.DMA,
            pltpu.SemaphoreType.DMA,
        ],
    )
    def k(idx_hbm, tb_hbm, te_hbm, gb_hbm, ge_hbm, idx_v, rb_v, re_v, semb, seme):
        wid = lax.axis_index("s") * _NC + lax.axis_index("c")
        base = wid * _ROWS_W
        pltpu.sync_copy(idx_hbm.at[pl.ds(base, _ROWS_W)], idx_v)

        def chunk(t, carry):
            off = t * _CHUNK
            ids = idx_v.at[pl.ds(off, _CHUNK)]
            cb = pltpu.async_copy(tb_hbm.at[ids], rb_v, semb)
            ce = pltpu.async_copy(te_hbm.at[ids], re_v, seme)
            cb.wait()
            pltpu.sync_copy(rb_v, gb_hbm.at[pl.ds(base + off, _CHUNK)])
            ce.wait()
            pltpu.sync_copy(re_v, ge_hbm.at[pl.ds(base + off, _CHUNK)])
            return carry

        lax.fori_loop(0, _NCHUNK, chunk, 0)

    return k(loc_ind, tb, te)


_R_BLK = 4096
_R_GRID = N // _R_BLK


def _tail_body(gb, ge, bf, ef, csm, alert, eff_out, base_out, outc_out):
    blin = jnp.sum(gb[...][:, :DB] * bf[...], axis=1).reshape(1, _R_BLK)
    elin = jnp.sum(ge[...][:, :DE] * ef[...], axis=1).reshape(1, _R_BLK) - 4.0
    baseline = jnp.minimum(jnp.exp(blin), 1e6)
    eff = jnp.clip(jax.nn.sigmoid(elin), 1e-6, 1.0 - 1e-6)
    eff_out[...] = eff
    base_out[...] = baseline
    outc_out[...] = csm[...] * baseline * (1.0 - alert[...] * eff)


def _tail(gb, ge, bf, ef, csm, alert):
    row = lambda i: (i, 0)
    csm2 = csm.reshape(_R_GRID, _R_BLK)
    alert2 = alert.reshape(_R_GRID, _R_BLK)
    return pl.pallas_call(
        _tail_body,
        grid=(_R_GRID,),
        in_specs=[
            pl.BlockSpec((_R_BLK, CP), row),
            pl.BlockSpec((_R_BLK, CP), row),
            pl.BlockSpec((_R_BLK, DB), row),
            pl.BlockSpec((_R_BLK, DE), row),
            pl.BlockSpec((1, _R_BLK), row),
            pl.BlockSpec((1, _R_BLK), row),
        ],
        out_specs=[
            pl.BlockSpec((1, _R_BLK), row),
            pl.BlockSpec((1, _R_BLK), row),
            pl.BlockSpec((1, _R_BLK), row),
        ],
        out_shape=[
            jax.ShapeDtypeStruct((_R_GRID, _R_BLK), jnp.float32),
            jax.ShapeDtypeStruct((_R_GRID, _R_BLK), jnp.float32),
            jax.ShapeDtypeStruct((_R_GRID, _R_BLK), jnp.float32),
        ],
    )(gb, ge, bf, ef, csm2, alert2)


def kernel(hosps, loc_ind, county_summer_mean, alert, baseline_features,
           eff_features, index, spatial_features,
           Wb1, bb1, Wb2, bb2, We1, be1, We2, be2):
    tb, te = _mlp_tables(spatial_features, Wb1, bb1, Wb2, bb2,
                         We1, be1, We2, be2)
    gb, ge = _sc_gather(loc_ind, tb, te)
    eff, base, outc = _tail(gb, ge, baseline_features, eff_features,
                            county_summer_mean, alert)
    return jnp.stack([eff.reshape(N), base.reshape(N), outc.reshape(N)],
                     axis=1)


# speedup vs baseline: 2.4590x; 2.4590x over previous
import functools

import jax
import jax.numpy as jnp
from jax import lax
from jax.experimental import pallas as pl
from jax.experimental.pallas import tpu as pltpu
from jax.experimental.pallas import tpu_sc as plsc

S = 100000
DS = 32
N = 524288
DB = 26
DE = 26
H = 32
CP = 32


_S_BLK = 2000


def _mlp_body(sf, wb1, bb1, wb2, bb2, we1, be1, we2, be2, tb_out, te_out):
    x = sf[...]
    hb = jax.nn.silu(jnp.dot(x, wb1[...], preferred_element_type=jnp.float32)
                     + bb1[...])
    tb_out[...] = (jnp.dot(hb, wb2[...], preferred_element_type=jnp.float32)
                   + bb2[...])
    he = jax.nn.silu(jnp.dot(x, we1[...], preferred_element_type=jnp.float32)
                     + be1[...])
    te_out[...] = (jnp.dot(he, we2[...], preferred_element_type=jnp.float32)
                   + be2[...])


def _mlp_tables(sf, Wb1, bb1, Wb2, bb2, We1, be1, We2, be2):
    Wb2p = jnp.pad(Wb2, ((0, 0), (0, CP - DB)))
    bb2p = jnp.pad(bb2, (0, CP - DB)).reshape(1, CP)
    We2p = jnp.pad(We2, ((0, 0), (0, CP - DE)))
    be2p = jnp.pad(be2, (0, CP - DE)).reshape(1, CP)
    bb1r = bb1.reshape(1, H)
    be1r = be1.reshape(1, H)

    grid = S // _S_BLK
    full = lambda i: (0, 0)
    return pl.pallas_call(
        _mlp_body,
        grid=(grid,),
        in_specs=[
            pl.BlockSpec((_S_BLK, DS), lambda i: (i, 0)),
            pl.BlockSpec((DS, H), full),
            pl.BlockSpec((1, H), full),
            pl.BlockSpec((H, CP), full),
            pl.BlockSpec((1, CP), full),
            pl.BlockSpec((DS, H), full),
            pl.BlockSpec((1, H), full),
            pl.BlockSpec((H, CP), full),
            pl.BlockSpec((1, CP), full),
        ],
        out_specs=[
            pl.BlockSpec((_S_BLK, CP), lambda i: (i, 0)),
            pl.BlockSpec((_S_BLK, CP), lambda i: (i, 0)),
        ],
        out_shape=[
            jax.ShapeDtypeStruct((S, CP), jnp.float32),
            jax.ShapeDtypeStruct((S, CP), jnp.float32),
        ],
    )(sf, Wb1, bb1r, Wb2p, bb2p, We1, be1r, We2p, be2p)


_NC = 2
_NS = 16
_NW = _NC * _NS
_ROWS_W = N // _NW
_CHUNK = 1024
_NCHUNK = _ROWS_W // _CHUNK


def _sc_gather(loc_ind, tb, te):
    mesh = plsc.VectorSubcoreMesh(core_axis_name="c", subcore_axis_name="s")

    @functools.partial(
        pl.kernel,
        mesh=mesh,
        out_type=(
            jax.ShapeDtypeStruct((N, CP), jnp.float32),
            jax.ShapeDtypeStruct((N, CP), jnp.float32),
        ),
        scratch_types=[
            pltpu.VMEM((_ROWS_W,), jnp.int32),
            pltpu.VMEM((_CHUNK, CP), jnp.float32),
            pltpu.VMEM((_CHUNK, CP), jnp.float32),
            pltpu.SemaphoreType.DMA,
            pltpu.SemaphoreType.DMA,
        ],
        compiler_params=pltpu.CompilerParams(use_tc_tiling_on_sc=False),
    )
    def k(idx_hbm, tb_hbm, te_hbm, gb_hbm, ge_hbm, idx_v, rb_v, re_v, semb, seme):
        wid = lax.axis_index("s") * _NC + lax.axis_index("c")
        base = wid * _ROWS_W
        pltpu.sync_copy(idx_hbm.at[pl.ds(base, _ROWS_W)], idx_v)

        def chunk(t, carry):
            off = t * _CHUNK
            ids = idx_v.at[pl.ds(off, _CHUNK)]
            cb = pltpu.async_copy(tb_hbm.at[ids], rb_v, semb)
            ce = pltpu.async_copy(te_hbm.at[ids], re_v, seme)
            cb.wait()
            pltpu.sync_copy(rb_v, gb_hbm.at[pl.ds(base + off, _CHUNK)])
            ce.wait()
            pltpu.sync_copy(re_v, ge_hbm.at[pl.ds(base + off, _CHUNK)])
            return carry

        lax.fori_loop(0, _NCHUNK, chunk, 0)

    return k(loc_ind, tb, te)


_R_BLK = 4096
_R_GRID = N // _R_BLK


def _tail_body(gb, ge, bf, ef, csm, alert, eff_out, base_out, outc_out):
    blin = jnp.sum(gb[...][:, :DB] * bf[...], axis=1).reshape(1, 1, _R_BLK)
    elin = jnp.sum(ge[...][:, :DE] * ef[...], axis=1).reshape(1, 1, _R_BLK) - 4.0
    baseline = jnp.minimum(jnp.exp(blin), 1e6)
    eff = jnp.clip(jax.nn.sigmoid(elin), 1e-6, 1.0 - 1e-6)
    eff_out[...] = eff
    base_out[...] = baseline
    outc_out[...] = csm[...] * baseline * (1.0 - alert[...] * eff)


def _tail(gb, ge, bf, ef, csm, alert):
    row = lambda i: (i, 0)
    row3 = lambda i: (i, 0, 0)
    csm2 = csm.reshape(_R_GRID, 1, _R_BLK)
    alert2 = alert.reshape(_R_GRID, 1, _R_BLK)
    return pl.pallas_call(
        _tail_body,
        grid=(_R_GRID,),
        in_specs=[
            pl.BlockSpec((_R_BLK, CP), row),
            pl.BlockSpec((_R_BLK, CP), row),
            pl.BlockSpec((_R_BLK, DB), row),
            pl.BlockSpec((_R_BLK, DE), row),
            pl.BlockSpec((1, 1, _R_BLK), row3),
            pl.BlockSpec((1, 1, _R_BLK), row3),
        ],
        out_specs=[
            pl.BlockSpec((1, 1, _R_BLK), row3),
            pl.BlockSpec((1, 1, _R_BLK), row3),
            pl.BlockSpec((1, 1, _R_BLK), row3),
        ],
        out_shape=[
            jax.ShapeDtypeStruct((_R_GRID, 1, _R_BLK), jnp.float32),
            jax.ShapeDtypeStruct((_R_GRID, 1, _R_BLK), jnp.float32),
            jax.ShapeDtypeStruct((_R_GRID, 1, _R_BLK), jnp.float32),
        ],
    )(gb, ge, bf, ef, csm2, alert2)


def kernel(hosps, loc_ind, county_summer_mean, alert, baseline_features,
           eff_features, index, spatial_features,
           Wb1, bb1, Wb2, bb2, We1, be1, We2, be2):
    tb, te = _mlp_tables(spatial_features, Wb1, bb1, Wb2, bb2,
                         We1, be1, We2, be2)
    gb, ge = _sc_gather(loc_ind, tb, te)
    eff, base, outc = _tail(gb, ge, baseline_features, eff_features,
                            county_summer_mean, alert)
    return jnp.stack([eff.reshape(N), base.reshape(N), outc.reshape(N)],
                     axis=1)
